# Initial kernel scaffold; baseline (speedup 1.0000x reference)
import functools

import jax
import jax.numpy as jnp
from jax import lax
from jax.experimental import pallas as pl
from jax.experimental.pallas import tpu as pltpu
from jax.experimental.pallas import tpu_sc as plsc

N = 100000
E = 3200000
D = 128

NC = 2
NS = 16
LANES = 16

K = 8
BS = 12544
NPAD = K * BS
ACC_ROWS = BS + 256
TRASH = BS
GB = 512
NB = GB // 128
CBUF = GB + 16
CB = 2000
EPT = E // NS
NBLK = EPT // CB
PASSES = K // NC
WPT = BS // NS
RPT = ACC_ROWS // NS
ZCH = 100

SB = 256
EPTS = 100096
EPAD = NC * NS * EPTS
NBATCH = EPTS // SB

_MESH = plsc.VectorSubcoreMesh(core_axis_name="c", subcore_axis_name="s",
                               num_cores=NC, num_subcores=NS)


def _make_layer(with_deg):
  out_type = [jax.ShapeDtypeStruct((NPAD, D), jnp.float32)]
  if with_deg:
    out_type.append(jax.ShapeDtypeStruct((NPAD,), jnp.float32))
  scratch = [
      pltpu.VMEM_SHARED((ACC_ROWS, D), jnp.float32),
      pltpu.VMEM_SHARED((ACC_ROWS,), jnp.float32),
      pltpu.VMEM((CBUF,), jnp.int32),
      pltpu.VMEM((CBUF,), jnp.int32),
      pltpu.VMEM((NB, 128), jnp.int32),
      pltpu.VMEM((NB, 128), jnp.int32),
      pltpu.VMEM((GB, D), jnp.float32),
      pltpu.VMEM((128,), jnp.float32),
      pltpu.VMEM((ZCH, D), jnp.float32),
      pltpu.VMEM((RPT,), jnp.float32),
      pltpu.VMEM((CB,), jnp.int32),
      pltpu.VMEM((CB,), jnp.int32),
      pltpu.SemaphoreType.DMA,
  ]

  @functools.partial(pl.kernel, mesh=_MESH, out_type=tuple(out_type),
                     scratch_types=scratch)
  def kern(y_hbm, src_hbm, dst_hbm, *rest):
    if with_deg:
      agg_hbm, deg_hbm = rest[0], rest[1]
      (acc, dacc, cb_src, cb_ldst, gidx, lidx, rows, ones, zbuf, dzbuf,
       sbuf, dbuf, sem) = rest[2:]
    else:
      agg_hbm, deg_hbm = rest[0], None
      (acc, dacc, cb_src, cb_ldst, gidx, lidx, rows, ones, zbuf, dzbuf,
       sbuf, dbuf, sem) = rest[1:]

    cid = lax.axis_index("c")
    sid = lax.axis_index("s")
    lane = lax.iota(jnp.int32, LANES)
    zv = jnp.zeros((LANES,), jnp.float32)
    onev = jnp.full((LANES,), 1.0, jnp.float32)
    ebase = sid * EPT

    def _zrow(i, c):
      for k8 in range(D // LANES):
        zbuf[i, pl.ds(k8 * LANES, LANES)] = zv
      return c
    lax.fori_loop(0, ZCH, _zrow, 0)

    def _zd(i, c):
      dzbuf[pl.ds(i * LANES, LANES)] = zv
      return c
    lax.fori_loop(0, RPT // LANES, _zd, 0)

    for k8 in range(128 // LANES):
      ones[pl.ds(k8 * LANES, LANES)] = onev

    def _flush():
      for jj in range(GB // LANES):
        r, c0 = jj // 8, (jj % 8) * LANES
        gidx[r, pl.ds(c0, LANES)] = cb_src[pl.ds(jj * LANES, LANES)]
        lidx[r, pl.ds(c0, LANES)] = cb_ldst[pl.ds(jj * LANES, LANES)]
      cps = [pltpu.async_copy(y_hbm.at[gidx.at[j]],
                              rows.at[pl.ds(j * 128, 128)], sem)
             for j in range(NB)]
      for c in cps:
        c.wait()
      for j in range(NB):
        pltpu.sync_copy(rows.at[pl.ds(j * 128, 128)], acc.at[lidx.at[j]],
                        add=True)
        if with_deg:
          pltpu.sync_copy(ones, dacc.at[lidx.at[j]], add=True)

    def _pass(p, carry):
      b = p * NC + cid
      base = b * BS
      plsc.subcore_barrier()

      def _zcp(i, c):
        pltpu.sync_copy(zbuf, acc.at[pl.ds(sid * RPT + i * ZCH, ZCH)])
        return c
      lax.fori_loop(0, RPT // ZCH, _zcp, 0)
      if with_deg:
        pltpu.sync_copy(dzbuf, dacc.at[pl.ds(sid * RPT, RPT)])
      plsc.subcore_barrier()

      def _blk(i, cur):
        pltpu.sync_copy(src_hbm.at[pl.ds(ebase + i * CB, CB)], sbuf)
        pltpu.sync_copy(dst_hbm.at[pl.ds(ebase + i * CB, CB)], dbuf)

        def _vr(j, cur):
          s = sbuf[pl.ds(j * LANES, LANES)]
          d = dbuf[pl.ds(j * LANES, LANES)]
          m = (d >= base) & (d < base + BS)
          plsc.store_compressed(cb_src.at[pl.ds(cur, LANES)], s, mask=m)
          plsc.store_compressed(cb_ldst.at[pl.ds(cur, LANES)], d - base,
                                mask=m)
          cur = cur + jnp.sum(m.astype(jnp.int32))

          def _do_flush(c):
            _flush()
            cb_src[pl.ds(0, LANES)] = cb_src[pl.ds(GB, LANES)]
            cb_ldst[pl.ds(0, LANES)] = cb_ldst[pl.ds(GB, LANES)]
            return c - GB
          return lax.cond(cur >= GB, _do_flush, lambda c: c, cur)
        return lax.fori_loop(0, CB // LANES, _vr, cur)
      cur = lax.fori_loop(0, NBLK, _blk, jnp.int32(0))

      for jj in range(CBUF // LANES):
        pm = (lane + jj * LANES) >= cur
        sv = cb_src[pl.ds(jj * LANES, LANES)]
        lv = cb_ldst[pl.ds(jj * LANES, LANES)]
        cb_src[pl.ds(jj * LANES, LANES)] = jnp.where(pm, 0, sv)
        cb_ldst[pl.ds(jj * LANES, LANES)] = jnp.where(pm, TRASH, lv)
      _flush()

      plsc.subcore_barrier()
      pltpu.sync_copy(acc.at[pl.ds(sid * WPT, WPT)],
                      agg_hbm.at[pl.ds(base + sid * WPT, WPT)])
      if with_deg:
        pltpu.sync_copy(dacc.at[pl.ds(sid * WPT, WPT)],
                        deg_hbm.at[pl.ds(base + sid * WPT, WPT)])
      return carry
    lax.fori_loop(0, PASSES, _pass, 0)
  return kern


def _make_score():
  scratch = [
      pltpu.VMEM((2, 128), jnp.int32),
      pltpu.VMEM((2, 128), jnp.int32),
      pltpu.VMEM((SB, D), jnp.float32),
      pltpu.VMEM((SB, D), jnp.float32),
      pltpu.VMEM((SB,), jnp.float32),
      pltpu.SemaphoreType.DMA,
  ]

  @functools.partial(
      pl.kernel, mesh=_MESH,
      out_type=jax.ShapeDtypeStruct((EPAD,), jnp.float32),
      scratch_types=scratch)
  def kern(h_hbm, srcp_hbm, dstp_hbm, out_hbm, sidx, didx, arows, brows,
           scores, sem):
    cid = lax.axis_index("c")
    sid = lax.axis_index("s")
    t = cid * NS + sid
    lane = lax.iota(jnp.int32, LANES)
    rowbase = t * (EPTS // 128)

    def _batch(ib, carry):
      pltpu.sync_copy(srcp_hbm.at[pl.ds(rowbase + ib * (SB // 128),
                                        SB // 128)], sidx)
      pltpu.sync_copy(dstp_hbm.at[pl.ds(rowbase + ib * (SB // 128),
                                        SB // 128)], didx)
      cps = [pltpu.async_copy(h_hbm.at[sidx.at[j]],
                              arows.at[pl.ds(j * 128, 128)], sem)
             for j in range(SB // 128)]
      cps += [pltpu.async_copy(h_hbm.at[didx.at[j]],
                               brows.at[pl.ds(j * 128, 128)], sem)
              for j in range(SB // 128)]
      for c in cps:
        c.wait()

      def _grp(g, carry):
        sv = jnp.zeros((LANES,), jnp.float32)
        for i in range(LANES):
          e = g * LANES + i
          a0 = arows[e, pl.ds(0, LANES)] * brows[e, pl.ds(0, LANES)]
          for kk in range(1, D // LANES):
            a0 = a0 + (arows[e, pl.ds(kk * LANES, LANES)] *
                       brows[e, pl.ds(kk * LANES, LANES)])
          sv = jnp.where(lane == i, jnp.sum(a0), sv)
        scores[pl.ds(g * LANES, LANES)] = sv
        return carry
      lax.fori_loop(0, SB // LANES, _grp, 0)
      pltpu.sync_copy(scores, out_hbm.at[pl.ds(t * EPTS + ib * SB, SB)])
      return carry
    lax.fori_loop(0, NBATCH, _batch, 0)
  return kern


_layer_deg = _make_layer(True)
_layer = _make_layer(False)
_score = _make_score()

BR = 1000
BR2 = 1024


def _mm_body(x_ref, w_ref, o_ref):
  o_ref[...] = jnp.dot(x_ref[...], w_ref[...],
                       preferred_element_type=jnp.float32)


def _tc_matmul(x, w):
  return pl.pallas_call(
      _mm_body,
      grid=(N // BR,),
      in_specs=[pl.BlockSpec((BR, D), lambda i: (i, 0)),
                pl.BlockSpec((D, D), lambda i: (0, 0))],
      out_specs=pl.BlockSpec((BR, D), lambda i: (i, 0)),
      out_shape=jax.ShapeDtypeStruct((N, D), jnp.float32),
  )(x, w)


def _l2_body(agg_ref, deg_ref, w_ref, o_ref):
  inv = 1.0 / jnp.maximum(deg_ref[...], 1.0)
  h = jnp.maximum(agg_ref[...] * inv.reshape(BR2, 1), 0.0)
  o_ref[...] = jnp.dot(h, w_ref[...], preferred_element_type=jnp.float32)


def _tc_l2(agg, deg2d, w):
  return pl.pallas_call(
      _l2_body,
      grid=(NPAD // BR2,),
      in_specs=[pl.BlockSpec((BR2, D), lambda i: (i, 0)),
                pl.BlockSpec((BR2 // 128, 128), lambda i: (i, 0)),
                pl.BlockSpec((D, D), lambda i: (0, 0))],
      out_specs=pl.BlockSpec((BR2, D), lambda i: (i, 0)),
      out_shape=jax.ShapeDtypeStruct((NPAD, D), jnp.float32),
  )(agg, deg2d, w)


def _h2_body(agg_ref, deg_ref, o_ref):
  inv = 1.0 / jnp.maximum(deg_ref[...], 1.0)
  o_ref[...] = agg_ref[...] * inv.reshape(BR2, 1)


def _tc_h2(agg, deg2d):
  return pl.pallas_call(
      _h2_body,
      grid=(NPAD // BR2,),
      in_specs=[pl.BlockSpec((BR2, D), lambda i: (i, 0)),
                pl.BlockSpec((BR2 // 128, 128), lambda i: (i, 0))],
      out_specs=pl.BlockSpec((BR2, D), lambda i: (i, 0)),
      out_shape=jax.ShapeDtypeStruct((NPAD, D), jnp.float32),
  )(agg, deg2d)


@jax.jit
def kernel(x, edge_index, W1, W2):
  src = edge_index[0].astype(jnp.int32)
  dst = edge_index[1].astype(jnp.int32)
  y1 = _tc_matmul(x, W1)
  agg1, deg = _layer_deg(y1, src, dst)
  deg2d = deg.reshape(NPAD // 128, 128)
  y2 = _tc_l2(agg1, deg2d, W2)
  agg2 = _layer(y2, src, dst)
  h2 = _tc_h2(agg2, deg2d)
  pad = jnp.zeros((EPAD - E,), jnp.int32)
  srcp = jnp.concatenate([src, pad]).reshape(EPAD // 128, 128)
  dstp = jnp.concatenate([dst, pad]).reshape(EPAD // 128, 128)
  score = _score(h2, srcp, dstp)
  return score[:E]


# speedup vs baseline: 5.1485x; 5.1485x over previous
import functools

import jax
import jax.numpy as jnp
from jax import lax
from jax.experimental import pallas as pl
from jax.experimental.pallas import tpu as pltpu
from jax.experimental.pallas import tpu_sc as plsc

N = 100000
E = 3200000
D = 128

NC = 2
NS = 16
LANES = 16

K = 10
BS = 10112
NPAD = K * BS
ACC_ROWS = BS + 128
TRASH = BS
GB = 256
NB = GB // 128
CBUF = GB + 16
CB = 2000
EPT = E // NS
NBLK = EPT // CB
PASSES = K // NC
WPT = BS // NS
RPT = ACC_ROWS // NS
ZCH = 128

SB = 256
EPTS = 100096
EPAD = NC * NS * EPTS
NBATCH = EPTS // SB

_MESH = plsc.VectorSubcoreMesh(core_axis_name="c", subcore_axis_name="s",
                               num_cores=NC, num_subcores=NS)


def _make_layer(with_deg):
  out_type = [jax.ShapeDtypeStruct((NPAD, D), jnp.float32)]
  if with_deg:
    out_type.append(jax.ShapeDtypeStruct((NPAD,), jnp.float32))
  scratch = [
      pltpu.VMEM_SHARED((ACC_ROWS, D), jnp.float32),
      pltpu.VMEM_SHARED((ACC_ROWS,), jnp.float32),
      pltpu.VMEM((CBUF,), jnp.int32),
      pltpu.VMEM((CBUF,), jnp.int32),
      pltpu.VMEM((NB, 128), jnp.int32),
      pltpu.VMEM((NB, 128), jnp.int32),
      pltpu.VMEM((GB, D), jnp.float32),
      pltpu.VMEM((128,), jnp.float32),
      pltpu.VMEM((CB,), jnp.int32),
      pltpu.VMEM((CB,), jnp.int32),
      pltpu.VMEM((RPT,), jnp.float32),
      pltpu.SemaphoreType.DMA,
  ]

  @functools.partial(pl.kernel, mesh=_MESH, out_type=tuple(out_type),
                     compiler_params=pltpu.CompilerParams(
                         needs_layout_passes=False),
                     scratch_types=scratch)
  def kern(y_hbm, src_hbm, dst_hbm, *rest):
    if with_deg:
      agg_hbm, deg_hbm = rest[0], rest[1]
      (acc, dacc, cb_src, cb_ldst, gidx, lidx, rows, ones,
       sbuf, dbuf, dwb, sem) = rest[2:]
    else:
      agg_hbm, deg_hbm = rest[0], None
      (acc, dacc, cb_src, cb_ldst, gidx, lidx, rows, ones,
       sbuf, dbuf, dwb, sem) = rest[1:]

    cid = lax.axis_index("c")
    sid = lax.axis_index("s")
    lane = lax.iota(jnp.int32, LANES)
    zv = jnp.zeros((LANES,), jnp.float32)
    onev = jnp.full((LANES,), 1.0, jnp.float32)
    ebase = sid * EPT

    for k8 in range(128 // LANES):
      ones[pl.ds(k8 * LANES, LANES)] = onev

    def _flush():
      for jj in range(GB // LANES):
        r, c0 = jj // 8, (jj % 8) * LANES
        gidx[r, pl.ds(c0, LANES)] = cb_src[pl.ds(jj * LANES, LANES)]
        lidx[r, pl.ds(c0, LANES)] = cb_ldst[pl.ds(jj * LANES, LANES)]
      cps = [pltpu.async_copy(y_hbm.at[gidx.at[j]],
                              rows.at[pl.ds(j * 128, 128)], sem)
             for j in range(NB)]
      for c in cps:
        c.wait()
      for j in range(NB):
        pltpu.sync_copy(rows.at[pl.ds(j * 128, 128)], acc.at[lidx.at[j]],
                        add=True)
        if with_deg:
          pltpu.sync_copy(ones, dacc.at[lidx.at[j]], add=True)

    def _pass(p, carry):
      b = p * NC + cid
      base = b * BS
      plsc.subcore_barrier()

      def _zrow(i, c):
        for k8 in range(D // LANES):
          rows[i, pl.ds(k8 * LANES, LANES)] = zv
        return c
      lax.fori_loop(0, ZCH, _zrow, 0)

      def _zd(i, c):
        dwb[pl.ds(i * LANES, LANES)] = zv
        return c
      lax.fori_loop(0, RPT // LANES, _zd, 0)

      def _zcp(i, c):
        pltpu.sync_copy(rows.at[pl.ds(0, ZCH)],
                        acc.at[pl.ds(sid * RPT + i * ZCH, ZCH)])
        return c
      lax.fori_loop(0, RPT // ZCH, _zcp, 0)
      if with_deg:
        pltpu.sync_copy(dwb, dacc.at[pl.ds(sid * RPT, RPT)])
      plsc.subcore_barrier()

      def _blk(i, cur):
        pltpu.sync_copy(src_hbm.at[pl.ds(ebase + i * CB, CB)], sbuf)
        pltpu.sync_copy(dst_hbm.at[pl.ds(ebase + i * CB, CB)], dbuf)

        def _vr(j, cur):
          s = sbuf[pl.ds(j * LANES, LANES)]
          d = dbuf[pl.ds(j * LANES, LANES)]
          m = (d >= base) & (d < base + BS)
          mi = m.astype(jnp.int32)
          csum = plsc.cumsum(mi)
          pos = (cur - 1) + csum
          plsc.store_scatter(cb_src, [pos], s, mask=m)
          plsc.store_scatter(cb_ldst, [pos], d - base, mask=m)
          cur = cur + plsc.all_reduce_population_count(m)

          def _do_flush(c):
            _flush()
            cb_src[pl.ds(0, LANES)] = cb_src[pl.ds(GB, LANES)]
            cb_ldst[pl.ds(0, LANES)] = cb_ldst[pl.ds(GB, LANES)]
            return c - GB
          return lax.cond(jnp.all(cur >= GB), _do_flush, lambda c: c, cur)
        return lax.fori_loop(0, CB // LANES, _vr, cur)
      cur = lax.fori_loop(0, NBLK, _blk, jnp.zeros((LANES,), jnp.int32))

      for jj in range(CBUF // LANES):
        pm = (lane + jj * LANES) >= cur
        sv = cb_src[pl.ds(jj * LANES, LANES)]
        lv = cb_ldst[pl.ds(jj * LANES, LANES)]
        cb_src[pl.ds(jj * LANES, LANES)] = jnp.where(pm, 0, sv)
        cb_ldst[pl.ds(jj * LANES, LANES)] = jnp.where(pm, TRASH, lv)
      _flush()

      plsc.subcore_barrier()
      done = 0
      for ch in (GB, GB, WPT - 2 * GB):
        pltpu.sync_copy(acc.at[pl.ds(sid * WPT + done, ch)],
                        rows.at[pl.ds(0, ch)])
        pltpu.sync_copy(rows.at[pl.ds(0, ch)],
                        agg_hbm.at[pl.ds(base + sid * WPT + done, ch)])
        done += ch
      if with_deg:
        pltpu.sync_copy(dacc.at[pl.ds(sid * WPT, WPT)],
                        dwb.at[pl.ds(0, WPT)])
        pltpu.sync_copy(dwb.at[pl.ds(0, WPT)],
                        deg_hbm.at[pl.ds(base + sid * WPT, WPT)])
      return carry
    lax.fori_loop(0, PASSES, _pass, 0)
  return kern


def _make_score():
  scratch = [
      pltpu.VMEM((2, 128), jnp.int32),
      pltpu.VMEM((2, 128), jnp.int32),
      pltpu.VMEM((SB, D), jnp.float32),
      pltpu.VMEM((SB, D), jnp.float32),
      pltpu.VMEM((SB,), jnp.float32),
      pltpu.SemaphoreType.DMA,
  ]

  @functools.partial(
      pl.kernel, mesh=_MESH,
      out_type=jax.ShapeDtypeStruct((EPAD,), jnp.float32),
      compiler_params=pltpu.CompilerParams(needs_layout_passes=False),
      scratch_types=scratch)
  def kern(h_hbm, srcp_hbm, dstp_hbm, out_hbm, sidx, didx, arows, brows,
           scores, sem):
    cid = lax.axis_index("c")
    sid = lax.axis_index("s")
    t = cid * NS + sid
    lane = lax.iota(jnp.int32, LANES)
    rowbase = t * (EPTS // 128)

    def _batch(ib, carry):
      pltpu.sync_copy(srcp_hbm.at[pl.ds(rowbase + ib * (SB // 128),
                                        SB // 128)], sidx)
      pltpu.sync_copy(dstp_hbm.at[pl.ds(rowbase + ib * (SB // 128),
                                        SB // 128)], didx)
      cps = [pltpu.async_copy(h_hbm.at[sidx.at[j]],
                              arows.at[pl.ds(j * 128, 128)], sem)
             for j in range(SB // 128)]
      cps += [pltpu.async_copy(h_hbm.at[didx.at[j]],
                               brows.at[pl.ds(j * 128, 128)], sem)
              for j in range(SB // 128)]
      for c in cps:
        c.wait()

      m15 = lane == (LANES - 1)

      def _grp(g, carry):
        for i in range(LANES):
          e = g * LANES + i
          a0 = arows[e, pl.ds(0, LANES)] * brows[e, pl.ds(0, LANES)]
          for kk in range(1, D // LANES):
            a0 = a0 + (arows[e, pl.ds(kk * LANES, LANES)] *
                       brows[e, pl.ds(kk * LANES, LANES)])
          tot = plsc.cumsum(a0)
          ev = jnp.full((LANES,), 1, jnp.int32) * e
          plsc.store_scatter(scores, [ev], tot, mask=m15)
        return carry
      lax.fori_loop(0, SB // LANES, _grp, 0)
      pltpu.sync_copy(scores, out_hbm.at[pl.ds(t * EPTS + ib * SB, SB)])
      return carry
    lax.fori_loop(0, NBATCH, _batch, 0)
  return kern


_layer_deg = _make_layer(True)
_layer = _make_layer(False)
_score = _make_score()

BR = 1000
BR2 = 1024


def _mm_body(x_ref, w_ref, o_ref):
  o_ref[...] = jnp.dot(x_ref[...], w_ref[...],
                       preferred_element_type=jnp.float32)


def _tc_matmul(x, w):
  return pl.pallas_call(
      _mm_body,
      grid=(N // BR,),
      in_specs=[pl.BlockSpec((BR, D), lambda i: (i, 0)),
                pl.BlockSpec((D, D), lambda i: (0, 0))],
      out_specs=pl.BlockSpec((BR, D), lambda i: (i, 0)),
      out_shape=jax.ShapeDtypeStruct((N, D), jnp.float32),
  )(x, w)


def _l2_body(agg_ref, deg_ref, w_ref, o_ref):
  inv = 1.0 / jnp.maximum(deg_ref[...], 1.0)
  h = jnp.maximum(agg_ref[...] * inv, 0.0)
  o_ref[...] = jnp.dot(h, w_ref[...], preferred_element_type=jnp.float32)


def _tc_l2(agg, deg2d, w):
  return pl.pallas_call(
      _l2_body,
      grid=(NPAD // BR2,),
      in_specs=[pl.BlockSpec((BR2, D), lambda i: (i, 0)),
                pl.BlockSpec((BR2, 1), lambda i: (i, 0)),
                pl.BlockSpec((D, D), lambda i: (0, 0))],
      out_specs=pl.BlockSpec((BR2, D), lambda i: (i, 0)),
      out_shape=jax.ShapeDtypeStruct((NPAD, D), jnp.float32),
  )(agg, deg2d, w)


def _h2_body(agg_ref, deg_ref, o_ref):
  inv = 1.0 / jnp.maximum(deg_ref[...], 1.0)
  o_ref[...] = agg_ref[...] * inv


def _tc_h2(agg, deg2d):
  return pl.pallas_call(
      _h2_body,
      grid=(NPAD // BR2,),
      in_specs=[pl.BlockSpec((BR2, D), lambda i: (i, 0)),
                pl.BlockSpec((BR2, 1), lambda i: (i, 0))],
      out_specs=pl.BlockSpec((BR2, D), lambda i: (i, 0)),
      out_shape=jax.ShapeDtypeStruct((NPAD, D), jnp.float32),
  )(agg, deg2d)


@jax.jit
def kernel(x, edge_index, W1, W2):
  src = edge_index[0].astype(jnp.int32)
  dst = edge_index[1].astype(jnp.int32)
  y1 = _tc_matmul(x, W1)
  agg1, deg = _layer_deg(y1, src, dst)
  deg2d = deg.reshape(NPAD, 1)
  y2 = _tc_l2(agg1, deg2d, W2)
  (agg2,) = _layer(y2, src, dst)
  h2 = _tc_h2(agg2, deg2d)
  pad = jnp.zeros((EPAD - E,), jnp.int32)
  srcp = jnp.concatenate([src, pad]).reshape(EPAD // 128, 128)
  dstp = jnp.concatenate([dst, pad]).reshape(EPAD // 128, 128)
  score = _score(h2, srcp, dstp)
  return score[:E]
